# baseline (speedup 1.0000x reference)
import functools

import jax
import jax.numpy as jnp
from jax import lax
from jax.experimental import pallas as pl
from jax.experimental.pallas import tpu as pltpu
from jax.experimental.pallas import tpu_sc as plsc

B = 4096
L = 200
DIM = 64
N = B * L
NW = 32
PER_W = N // NW
CH = 1024
G = CH // 128
NCH = PER_W // CH
EPS = 1e-6

VOC = 1000000
TW = 4096
NQ = -(-VOC // (4 * TW))
Q = NQ * TW
VOC2 = 4 * Q

@functools.cache
def _make_gather_sc():
    mesh = plsc.VectorSubcoreMesh(core_axis_name="c", subcore_axis_name="s")

    @functools.partial(
        pl.kernel,
        mesh=mesh,
        compiler_params=pltpu.CompilerParams(use_tc_tiling_on_sc=False),
        out_type=jax.ShapeDtypeStruct((N // 4, 128), jnp.int32),
        scratch_types=[
            pltpu.VMEM((G, 128), jnp.int32),
            pltpu.VMEM((CH, DIM // 2), jnp.int32),
            pltpu.SemaphoreType.DMA,
        ],
    )
    def _gather_sc(idx_hbm, table_hbm, out_hbm, idx_v, rows_v, sem):
        wid = lax.axis_index("s") * 2 + lax.axis_index("c")
        base = wid * PER_W

        def chunk(ci, carry):
            off = pl.multiple_of(base + ci * CH, CH)
            row0 = pl.multiple_of(off // 128, G)
            pltpu.sync_copy(idx_hbm.at[pl.ds(row0, G)], idx_v)
            cps = [
                pltpu.async_copy(
                    table_hbm.at[idx_v.at[g]],
                    rows_v.at[pl.ds(g * 128, 128)],
                    sem,
                )
                for g in range(G)
            ]
            for cp in cps:
                cp.wait()
            r0 = pl.multiple_of((off >> 12) * (B // 4), CH)
            e = pl.multiple_of(((off >> 10) & 3) * (DIM // 2), DIM // 2)
            pltpu.sync_copy(rows_v, out_hbm.at[pl.ds(r0, CH), pl.ds(e, DIM // 2)])
            return carry

        lax.fori_loop(0, NCH, chunk, 0)

    return _gather_sc


def _rne(u):
    return u + jnp.uint32(0x7FFF) + ((u >> jnp.uint32(16)) & jnp.uint32(1))


def _bf16_pack(a):
    r = lax.bitcast_convert_type(a, jnp.uint32)
    tu = _rne(r[: DIM // 2, :])
    tv = _rne(r[DIM // 2:, :])
    pk = (tu >> jnp.uint32(16)) | (tv & jnp.uint32(0xFFFF0000))
    return lax.bitcast_convert_type(pk, jnp.int32)


def _tpack_body(a_ref, b_ref, c_ref, d_ref, o_ref):
    pk = jnp.concatenate(
        [_bf16_pack(r[...]) for r in (a_ref, b_ref, c_ref, d_ref)], axis=0
    )
    o_ref[...] = pk.T


def _tpack_tc(tT):
    specs = [
        pl.BlockSpec(
            (DIM, TW), functools.partial(lambda j, i: (0, jnp.minimum(j * NQ + i, VOC // TW)), j)
        )
        for j in range(4)
    ]
    return pl.pallas_call(
        _tpack_body,
        out_shape=jax.ShapeDtypeStruct((Q, 128), jnp.int32),
        grid=(NQ,),
        in_specs=specs,
        out_specs=pl.BlockSpec((TW, 128), lambda i: (i, 0)),
    )(tT, tT, tT, tT)


LB = 4


def _ln_body(word_ref, seg_ref, pos_ref, par_ref, o_ref):
    q = lax.bitcast_convert_type(word_ref[...], jnp.uint32)
    gT = par_ref[:, 0:1]
    bT = par_ref[:, 1:2]
    st0 = par_ref[:, 2:3]
    st1 = par_ref[:, 3:4]
    for i in range(LB):
        qiT = q[i * (B // 4):(i + 1) * (B // 4)].T
        parts = []
        for j in range(4):
            qjT = qiT[j * (DIM // 2):(j + 1) * (DIM // 2)]
            u = lax.bitcast_convert_type(qjT << jnp.uint32(16), jnp.float32)
            v = lax.bitcast_convert_type(qjT & jnp.uint32(0xFFFF0000), jnp.float32)
            parts.append(jnp.concatenate([u, v], axis=0))
        wt = jnp.concatenate(parts, axis=1)
        sf = seg_ref[i].astype(jnp.float32)
        posl = pos_ref[i]
        emb = wt + posl + st0 + sf * (st1 - st0)
        mean = jnp.mean(emb, axis=0, keepdims=True)
        ex2 = jnp.mean(emb * emb, axis=0, keepdims=True)
        var = ex2 - mean * mean
        normed = (emb - mean) * lax.rsqrt(var + EPS)
        o_ref[i] = normed * gT + bT


def _ln_tc(wordT, seg3, pos3, par):
    return pl.pallas_call(
        _ln_body,
        out_shape=jax.ShapeDtypeStruct((L, DIM, B), jnp.float32),
        grid=(L // LB,),
        in_specs=[
            pl.BlockSpec((LB * B // 4, 128), lambda l: (l, 0)),
            pl.BlockSpec((LB, 1, B), lambda l: (l, 0, 0)),
            pl.BlockSpec((LB, DIM, 1), lambda l: (l, 0, 0)),
            pl.BlockSpec((DIM, 4), lambda l: (0, 0)),
        ],
        out_specs=pl.BlockSpec((LB, DIM, B), lambda l: (l, 0, 0)),
    )(wordT, seg3, pos3, par)


def kernel(x, seg, table, segtable, posemb, gamma, beta):
    xT = x.T.astype(jnp.int32)
    xp = 4 * (xT % Q) + xT // Q
    idx2 = xp.reshape(N // 128, 128)
    table_lin = _tpack_tc(table.T).reshape(VOC2, DIM // 2)
    word2 = _make_gather_sc()(idx2, table_lin)
    seg3 = seg.T.reshape(L, 1, B)
    pos3 = posemb[:L].reshape(L, DIM, 1)
    par = jnp.stack([gamma, beta, segtable[0], segtable[1]], axis=1)
    outT = _ln_tc(word2, seg3, pos3, par)
    return outT.transpose(2, 0, 1)


# speedup vs baseline: 1.0028x; 1.0028x over previous
import functools

import jax
import jax.numpy as jnp
from jax import lax
from jax.experimental import pallas as pl
from jax.experimental.pallas import tpu as pltpu
from jax.experimental.pallas import tpu_sc as plsc

B = 4096
L = 200
DIM = 64
N = B * L
NW = 32
PER_W = N // NW
CH = 1024
G = CH // 128
NCH = PER_W // CH
EPS = 1e-6

VOC = 1000000
TW = 4096
NQ = -(-VOC // (4 * TW))
Q = NQ * TW
VOC2 = 4 * Q

@functools.cache
def _make_gather_sc():
    mesh = plsc.VectorSubcoreMesh(core_axis_name="c", subcore_axis_name="s")

    @functools.partial(
        pl.kernel,
        mesh=mesh,
        compiler_params=pltpu.CompilerParams(use_tc_tiling_on_sc=False),
        out_type=jax.ShapeDtypeStruct((N // 4, 128), jnp.int32),
        scratch_types=[
            pltpu.VMEM((G, 128), jnp.int32),
            pltpu.VMEM((CH, DIM // 2), jnp.int32),
            pltpu.SemaphoreType.DMA,
        ],
    )
    def _gather_sc(idx_hbm, table_hbm, out_hbm, idx_v, rows_v, sem):
        wid = lax.axis_index("s") * 2 + lax.axis_index("c")
        base = wid * PER_W

        def chunk(ci, carry):
            off = pl.multiple_of(base + ci * CH, CH)
            row0 = pl.multiple_of(off // 128, G)
            pltpu.sync_copy(idx_hbm.at[pl.ds(row0, G)], idx_v)
            cps = [
                pltpu.async_copy(
                    table_hbm.at[idx_v.at[g]],
                    rows_v.at[pl.ds(g * 128, 128)],
                    sem,
                )
                for g in range(G)
            ]
            for cp in cps:
                cp.wait()
            r0 = pl.multiple_of((off >> 12) * (B // 4), CH)
            e = pl.multiple_of(((off >> 10) & 3) * (DIM // 2), DIM // 2)
            pltpu.sync_copy(rows_v, out_hbm.at[pl.ds(r0, CH), pl.ds(e, DIM // 2)])
            return carry

        lax.fori_loop(0, NCH, chunk, 0)

    return _gather_sc


def _rne(u):
    return u + jnp.uint32(0x7FFF) + ((u >> jnp.uint32(16)) & jnp.uint32(1))


def _bf16_pack(a):
    r = lax.bitcast_convert_type(a, jnp.uint32)
    tu = _rne(r[: DIM // 2, :])
    tv = _rne(r[DIM // 2:, :])
    pk = (tu >> jnp.uint32(16)) | (tv & jnp.uint32(0xFFFF0000))
    return lax.bitcast_convert_type(pk, jnp.int32)


def _tpack_body(a_ref, b_ref, c_ref, d_ref, o_ref):
    pk = jnp.concatenate(
        [_bf16_pack(r[...]) for r in (a_ref, b_ref, c_ref, d_ref)], axis=0
    )
    o_ref[...] = pk.T


def _tpack_tc(tT):
    specs = [
        pl.BlockSpec(
            (DIM, TW), functools.partial(lambda j, i: (0, jnp.minimum(j * NQ + i, VOC // TW)), j)
        )
        for j in range(4)
    ]
    return pl.pallas_call(
        _tpack_body,
        out_shape=jax.ShapeDtypeStruct((Q, 128), jnp.int32),
        grid=(NQ,),
        in_specs=specs,
        out_specs=pl.BlockSpec((TW, 128), lambda i: (i, 0)),
    )(tT, tT, tT, tT)


LB = 8


def _ln_body(word_ref, seg_ref, pos_ref, par_ref, o_ref):
    q = lax.bitcast_convert_type(word_ref[...], jnp.uint32)
    gT = par_ref[:, 0:1]
    bT = par_ref[:, 1:2]
    st0 = par_ref[:, 2:3]
    st1 = par_ref[:, 3:4]
    for i in range(LB):
        qiT = q[i * (B // 4):(i + 1) * (B // 4)].T
        parts = []
        for j in range(4):
            qjT = qiT[j * (DIM // 2):(j + 1) * (DIM // 2)]
            u = lax.bitcast_convert_type(qjT << jnp.uint32(16), jnp.float32)
            v = lax.bitcast_convert_type(qjT & jnp.uint32(0xFFFF0000), jnp.float32)
            parts.append(jnp.concatenate([u, v], axis=0))
        wt = jnp.concatenate(parts, axis=1)
        sf = seg_ref[i].astype(jnp.float32)
        posl = pos_ref[i]
        emb = wt + posl + st0 + sf * (st1 - st0)
        mean = jnp.mean(emb, axis=0, keepdims=True)
        ex2 = jnp.mean(emb * emb, axis=0, keepdims=True)
        var = ex2 - mean * mean
        normed = (emb - mean) * lax.rsqrt(var + EPS)
        o_ref[i] = normed * gT + bT


def _ln_tc(wordT, seg3, pos3, par):
    return pl.pallas_call(
        _ln_body,
        out_shape=jax.ShapeDtypeStruct((L, DIM, B), jnp.float32),
        grid=(L // LB,),
        in_specs=[
            pl.BlockSpec((LB * B // 4, 128), lambda l: (l, 0)),
            pl.BlockSpec((LB, 1, B), lambda l: (l, 0, 0)),
            pl.BlockSpec((LB, DIM, 1), lambda l: (l, 0, 0)),
            pl.BlockSpec((DIM, 4), lambda l: (0, 0)),
        ],
        out_specs=pl.BlockSpec((LB, DIM, B), lambda l: (l, 0, 0)),
    )(wordT, seg3, pos3, par)


def kernel(x, seg, table, segtable, posemb, gamma, beta):
    xT = x.T.astype(jnp.int32)
    xp = 4 * (xT % Q) + xT // Q
    idx2 = xp.reshape(N // 128, 128)
    table_lin = _tpack_tc(table.T).reshape(VOC2, DIM // 2)
    word2 = _make_gather_sc()(idx2, table_lin)
    seg3 = seg.T.reshape(L, 1, B)
    pos3 = posemb[:L].reshape(L, DIM, 1)
    par = jnp.stack([gamma, beta, segtable[0], segtable[1]], axis=1)
    outT = _ln_tc(word2, seg3, pos3, par)
    return outT.transpose(2, 0, 1)
